# Initial kernel scaffold; baseline (speedup 1.0000x reference)
import jax
import jax.numpy as jnp
from jax.experimental import pallas as pl


def kernel(x, pos_embedding):
    raise NotImplementedError("write your pallas kernel here")


# speedup vs baseline: 7.3451x; 7.3451x over previous
import jax
import jax.numpy as jnp
from jax.experimental import pallas as pl


def _bcast_kernel(emb_ref, out_ref):
    out_ref[...] = jnp.broadcast_to(
        emb_ref[...][:, None, :], out_ref.shape
    )


def kernel(x, pos_embedding):
    S, N = x.shape
    D = pos_embedding.shape[1]
    BS = 512
    grid = (S // BS,)
    return pl.pallas_call(
        _bcast_kernel,
        grid=grid,
        in_specs=[pl.BlockSpec((BS, D), lambda i: (i, 0))],
        out_specs=pl.BlockSpec((BS, N, D), lambda i: (i, 0, 0)),
        out_shape=jax.ShapeDtypeStruct((S, N, D), pos_embedding.dtype),
    )(pos_embedding)
